# baseline (speedup 1.0000x reference)
import jax
import jax.numpy as jnp
from jax.experimental import pallas as pl


def _fused_kernel(feats_ref, adj_ref, w1_ref, b1_ref, aw_ref, wd_ref, bd_ref,
                  out_ref):
    x = feats_ref[0]
    xw = jnp.dot(x, w1_ref[...],
                 preferred_element_type=jnp.float32) + b1_ref[...]
    a = adj_ref[0]
    h = jnp.maximum(
        jnp.dot(a, xw, preferred_element_type=jnp.float32), 0.0)
    s = jnp.tanh(jnp.sum(h * aw_ref[...], axis=1, keepdims=True))
    m = jnp.max(s)
    e = jnp.exp(s - m)
    alpha = e / jnp.sum(e)
    rep = jnp.sum(alpha * h, axis=0, keepdims=True)
    out_ref[0] = jnp.dot(rep, wd_ref[...],
                         preferred_element_type=jnp.float32) + bd_ref[...]


def kernel(feats, adj, W1, b1, att_w, Wd, bd):
    B, N, F = feats.shape
    H = W1.shape[1]
    b1_2d = b1.reshape(1, H)
    aw_2d = att_w.reshape(1, H)
    bd_2d = bd.reshape(1, 1)
    out = pl.pallas_call(
        _fused_kernel,
        grid=(B,),
        in_specs=[
            pl.BlockSpec((1, N, F), lambda b: (b, 0, 0)),
            pl.BlockSpec((1, N, N), lambda b: (b, 0, 0)),
            pl.BlockSpec((F, H), lambda b: (0, 0)),
            pl.BlockSpec((1, H), lambda b: (0, 0)),
            pl.BlockSpec((1, H), lambda b: (0, 0)),
            pl.BlockSpec((H, 1), lambda b: (0, 0)),
            pl.BlockSpec((1, 1), lambda b: (0, 0)),
        ],
        out_specs=pl.BlockSpec((1, 1, 1), lambda b: (b, 0, 0)),
        out_shape=jax.ShapeDtypeStruct((B, 1, 1), jnp.float32),
    )(feats, adj, W1, b1_2d, aw_2d, Wd, bd_2d)
    return out[:, 0, 0]


# speedup vs baseline: 1.0680x; 1.0680x over previous
import functools

import jax
import jax.numpy as jnp
from jax import lax
from jax.experimental import pallas as pl

_G = 4


def _fused_kernel(feats_ref, adj_ref, w1_ref, b1_ref, aw_ref, wd_ref, bd_ref,
                  out_ref):
    for g in range(_G):
        x = feats_ref[g]
        xw = jnp.dot(x, w1_ref[...],
                     preferred_element_type=jnp.float32) + b1_ref[...]
        a = adj_ref[g]
        h = jnp.maximum(
            jnp.dot(a, xw, preferred_element_type=jnp.float32), 0.0)
        s = jnp.tanh(jnp.dot(h, aw_ref[...],
                             preferred_element_type=jnp.float32))
        m = jnp.max(s)
        e = jnp.exp(s - m)
        rep = lax.dot_general(e, h, (((0,), (0,)), ((), ())),
                              preferred_element_type=jnp.float32)
        denom = jnp.sum(e)
        out = jnp.dot(rep, wd_ref[...],
                      preferred_element_type=jnp.float32) / denom
        out_ref[g] = out + bd_ref[...]


def kernel(feats, adj, W1, b1, att_w, Wd, bd):
    B, N, F = feats.shape
    H = W1.shape[1]
    b1_2d = b1.reshape(1, H)
    aw_col = att_w.reshape(H, 1)
    bd_2d = bd.reshape(1, 1)
    out = pl.pallas_call(
        _fused_kernel,
        grid=(B // _G,),
        in_specs=[
            pl.BlockSpec((_G, N, F), lambda b: (b, 0, 0)),
            pl.BlockSpec((_G, N, N), lambda b: (b, 0, 0)),
            pl.BlockSpec((F, H), lambda b: (0, 0)),
            pl.BlockSpec((1, H), lambda b: (0, 0)),
            pl.BlockSpec((H, 1), lambda b: (0, 0)),
            pl.BlockSpec((H, 1), lambda b: (0, 0)),
            pl.BlockSpec((1, 1), lambda b: (0, 0)),
        ],
        out_specs=pl.BlockSpec((_G, 1, 1), lambda b: (b, 0, 0)),
        out_shape=jax.ShapeDtypeStruct((B, 1, 1), jnp.float32),
    )(feats, adj, W1, b1_2d, aw_col, Wd, bd_2d)
    return out[:, 0, 0]


# speedup vs baseline: 1.2285x; 1.1502x over previous
import functools

import jax
import jax.numpy as jnp
from jax import lax
from jax.experimental import pallas as pl

_G = 4


def _fused_kernel(feats_ref, adj_ref, w1_ref, b1_ref, aw_ref, wd_ref, bd_ref,
                  out_ref):
    for g in range(_G):
        x = feats_ref[g]
        xw = jnp.dot(x, w1_ref[...],
                     preferred_element_type=jnp.float32) + b1_ref[...]
        a = adj_ref[g]
        h = jnp.maximum(
            jnp.dot(a.astype(jnp.bfloat16), xw.astype(jnp.bfloat16),
                    preferred_element_type=jnp.float32), 0.0)
        s = jnp.tanh(jnp.dot(h, aw_ref[...],
                             preferred_element_type=jnp.float32))
        e = jnp.exp(s)
        rep = lax.dot_general(e, h, (((0,), (0,)), ((), ())),
                              preferred_element_type=jnp.float32)
        denom = jnp.sum(e)
        out = jnp.dot(rep, wd_ref[...],
                      preferred_element_type=jnp.float32) / denom
        out_ref[g] = out + bd_ref[...]


def kernel(feats, adj, W1, b1, att_w, Wd, bd):
    B, N, F = feats.shape
    H = W1.shape[1]
    b1_2d = b1.reshape(1, H)
    aw_col = att_w.reshape(H, 1)
    bd_2d = bd.reshape(1, 1)
    out = pl.pallas_call(
        _fused_kernel,
        grid=(B // _G,),
        in_specs=[
            pl.BlockSpec((_G, N, F), lambda b: (b, 0, 0)),
            pl.BlockSpec((_G, N, N), lambda b: (b, 0, 0)),
            pl.BlockSpec((F, H), lambda b: (0, 0)),
            pl.BlockSpec((1, H), lambda b: (0, 0)),
            pl.BlockSpec((H, 1), lambda b: (0, 0)),
            pl.BlockSpec((H, 1), lambda b: (0, 0)),
            pl.BlockSpec((1, 1), lambda b: (0, 0)),
        ],
        out_specs=pl.BlockSpec((_G, 1, 1), lambda b: (b, 0, 0)),
        out_shape=jax.ShapeDtypeStruct((B, 1, 1), jnp.float32),
    )(feats, adj, W1, b1_2d, aw_col, Wd, bd_2d)
    return out[:, 0, 0]


# speedup vs baseline: 1.8057x; 1.4699x over previous
import jax
import jax.numpy as jnp
from jax import lax
from jax.experimental import pallas as pl

_G = 4


def _fused_kernel(feats_ref, adj_ref, w1_ref, b1_ref, wsp_ref, bd_ref,
                  out_ref):
    N = adj_ref.shape[1]
    H = w1_ref.shape[1]
    f2d = feats_ref[...].reshape(_G * N, -1)
    xw = (jnp.dot(f2d.astype(jnp.bfloat16), w1_ref[...],
                  preferred_element_type=jnp.float32) + b1_ref[...])
    xwb = xw.astype(jnp.bfloat16)
    hs = []
    for g in range(_G):
        a = adj_ref[g].astype(jnp.bfloat16)
        h = jnp.maximum(
            jnp.dot(a, xwb[g * N:(g + 1) * N],
                    preferred_element_type=jnp.float32), 0)
        hs.append(h.astype(jnp.bfloat16))
    hcat = jnp.concatenate(hs, axis=1)
    sp = jnp.dot(hcat, wsp_ref[...],
                 preferred_element_type=jnp.float32)
    s = sp[:, :_G]
    p = sp[:, _G:2 * _G] + sp[:, 2 * _G:]
    e = jnp.exp(jnp.tanh(s))
    numer = jnp.sum(e * p, axis=0, keepdims=True)
    den = jnp.sum(e, axis=0, keepdims=True)
    out_ref[0] = numer / den + bd_ref[...]


def kernel(feats, adj, W1, b1, att_w, Wd, bd):
    B, N, F = feats.shape
    H = W1.shape[1]
    bf = jnp.bfloat16
    wd = Wd[:, 0]
    wd_hi = wd.astype(bf).astype(jnp.float32)
    wd_lo = wd - wd_hi
    eye = jnp.eye(_G, dtype=jnp.float32)
    blk = jnp.concatenate([
        jnp.einsum('h,gk->ghk', att_w, eye),
        jnp.einsum('h,gk->ghk', wd_hi, eye),
        jnp.einsum('h,gk->ghk', wd_lo, eye),
    ], axis=2)
    wsp = blk.reshape(_G * H, 3 * _G).astype(bf)
    out = pl.pallas_call(
        _fused_kernel,
        grid=(B // _G,),
        in_specs=[
            pl.BlockSpec((_G, N, F), lambda b: (b, 0, 0)),
            pl.BlockSpec((_G, N, N), lambda b: (b, 0, 0)),
            pl.BlockSpec((F, H), lambda b: (0, 0)),
            pl.BlockSpec((1, H), lambda b: (0, 0)),
            pl.BlockSpec((_G * H, 3 * _G), lambda b: (0, 0)),
            pl.BlockSpec((1, 1), lambda b: (0, 0)),
        ],
        out_specs=pl.BlockSpec((1, 1, _G), lambda b: (b, 0, 0)),
        out_shape=jax.ShapeDtypeStruct((B // _G, 1, _G), jnp.float32),
    )(feats, adj, W1.astype(bf), b1.reshape(1, H), wsp, bd.reshape(1, 1))
    return out.reshape(B)
